# baseline (speedup 1.0000x reference)
import functools

import jax
import jax.numpy as jnp
from jax import lax
from jax.experimental import pallas as pl
from jax.experimental.pallas import tpu as pltpu
from jax.experimental.pallas import tpu_sc as plsc

_EPS = 1e-4

_B, _N, _H, _W = 16, 1024, 128, 128
_HW = _H * _W
_NC, _NS, _L = 2, 16, 16
_NW = _NC * _NS
_KC = _N * 4

_mesh = plsc.VectorSubcoreMesh(core_axis_name="c", subcore_axis_name="s")


@functools.partial(
    pl.kernel,
    mesh=_mesh,
    out_type=jax.ShapeDtypeStruct((_NW * _L,), jnp.float32),
    compiler_params=pltpu.CompilerParams(needs_layout_passes=False),
    scratch_types=[
        pltpu.VMEM((_HW,), jnp.float32),
        pltpu.VMEM((_HW,), jnp.float32),
        pltpu.VMEM((_KC,), jnp.int32),
        pltpu.VMEM((_KC,), jnp.int32),
        pltpu.VMEM((8, 128), jnp.int32),
        pltpu.VMEM((2 * _N,), jnp.float32),
        pltpu.VMEM((8, 128), jnp.float32),
        pltpu.VMEM((_L,), jnp.float32),
        pltpu.SemaphoreType.DMA,
        pltpu.SemaphoreType.DMA,
    ],
)
def _sc_loss(coord_hbm, span_hbm, lc_hbm, ci_hbm, lcs_hbm, cm_hbm,
             out_hbm, coord_v, span_v, idx_v, gt_v, ci_v, lcs_v, cm_v,
             stage_v, sem, sem2):
    b = lax.axis_index("s")
    c = lax.axis_index("c")
    wid = b * _NC + c
    bt = lax.shift_right_logical(b, 3)
    bs = b & 7

    cp_span = [
        pltpu.async_copy(
            span_hbm.at[pl.ds((b * 2 + c) * _HW, _HW)], span_v, sem2),
        pltpu.async_copy(lcs_hbm.at[pl.ds(b * 2 * _N, 2 * _N)], lcs_v, sem2),
        pltpu.async_copy(ci_hbm.at[bt, :, bs], ci_v, sem2),
    ]
    cp_coord = [
        pltpu.async_copy(
            coord_hbm.at[pl.ds((b * 2 + c) * _HW, _HW)], coord_v, sem),
        pltpu.async_copy(lc_hbm.at[pl.ds(b * 3 * _KC, _KC)], idx_v, sem),
        pltpu.async_copy(
            lc_hbm.at[pl.ds((b * 3 + 1 + c) * _KC, _KC)], gt_v, sem),
        pltpu.async_copy(cm_hbm.at[bt, :, bs], cm_v, sem),
    ]
    for cp in cp_coord:
        cp.wait()

    @plsc.parallel_loop(0, _KC // _L, carry=jnp.zeros((_L,), jnp.float32),
                        unroll=4)
    def acc_c(g, acc):
        p0 = g * _L
        idx = idx_v[pl.ds(p0, _L)]
        gt = gt_v[pl.ds(p0, _L)].astype(jnp.float32)
        pred = plsc.load_gather(coord_v, [idx])
        m = cm_v[lax.shift_right_logical(g, 5), pl.ds((g & 7) * _L, _L)]
        return acc + jnp.abs(pred * m - gt * m)

    msel = jnp.where(c == 0, 1.0, 0.0).astype(jnp.float32)

    for cp in cp_span:
        cp.wait()

    @plsc.parallel_loop(
        0, _N // _L,
        carry=(jnp.zeros((_L,), jnp.float32), jnp.zeros((_L,), jnp.float32)),
        unroll=4)
    def span_accs(g, carry):
        acc_s, acc_m = carry
        nt = lax.shift_right_logical(g, 3)
        nl0 = (g & 7) * _L
        ind = ci_v[nt, pl.ds(nl0, _L)]
        sp = plsc.load_gather(span_v, [ind])
        gt = lcs_v[pl.ds(nt * 256 + c * 128 + nl0, _L)]
        m = cm_v[nt, pl.ds(nl0, _L)]
        return acc_s + jnp.abs(sp * m - gt * m), acc_m + m * msel

    acc_s, acc_m = span_accs

    lane = lax.iota(jnp.int32, _L)
    packed = jnp.where(lane == 0, jnp.sum(acc_c),
                       jnp.where(lane == 1, jnp.sum(acc_s),
                                 jnp.where(lane == 2, jnp.sum(acc_m), 0.0)))
    stage_v[pl.ds(0, _L)] = packed.astype(jnp.float32)
    pltpu.sync_copy(stage_v, out_hbm.at[pl.ds(wid * _L, _L)])


def kernel(coord, span, lc_coords, lc_span, ct_ind, ct_mask):
    lc_lin = (lc_coords.reshape(_B, 8, 128, 4, 3)
              .transpose(0, 4, 1, 3, 2)
              .reshape(_B * 3 * _KC))
    lcs_lin = (lc_span.reshape(_B, 8, 128, 2)
               .transpose(0, 1, 3, 2)
               .reshape(_B * 2 * _N))
    ci_lin = ct_ind.reshape(2, 8, 8, 128).transpose(0, 2, 1, 3)
    cm_lin = ct_mask.reshape(2, 8, 8, 128).transpose(0, 2, 1, 3)
    partials = _sc_loss(
        coord.reshape(_B * 2 * _HW),
        span.reshape(_B * 2 * _HW),
        lc_lin,
        ci_lin,
        lcs_lin,
        cm_lin,
    )
    s = partials.reshape(4, 128).sum(axis=0)
    coord_sum = s[0::_L].sum()
    span_sum = s[1::_L].sum()
    mask_sum = s[2::_L].sum()
    coord_loss = coord_sum / (4.0 * mask_sum + _EPS)
    span_loss = span_sum / (2.0 * mask_sum + _EPS)
    return (coord_loss, jnp.array(0.0, dtype=jnp.float32), span_loss)


# speedup vs baseline: 1.0460x; 1.0460x over previous
import functools

import jax
import jax.numpy as jnp
from jax import lax
from jax.experimental import pallas as pl
from jax.experimental.pallas import tpu as pltpu
from jax.experimental.pallas import tpu_sc as plsc

_EPS = 1e-4

_B, _N, _H, _W = 16, 1024, 128, 128
_HW = _H * _W
_NC, _NS, _L = 2, 16, 16
_NW = _NC * _NS
_KC = _N * 4

_mesh = plsc.VectorSubcoreMesh(core_axis_name="c", subcore_axis_name="s")


@functools.partial(
    pl.kernel,
    mesh=_mesh,
    out_type=jax.ShapeDtypeStruct((_NW * 128,), jnp.float32),
    compiler_params=pltpu.CompilerParams(needs_layout_passes=False),
    scratch_types=[
        pltpu.VMEM((_HW,), jnp.float32),
        pltpu.VMEM((_HW,), jnp.float32),
        pltpu.VMEM((_KC,), jnp.int32),
        pltpu.VMEM((_KC,), jnp.int32),
        pltpu.VMEM((8, 128), jnp.int32),
        pltpu.VMEM((2 * _N,), jnp.float32),
        pltpu.VMEM((8, 128), jnp.float32),
        pltpu.VMEM((128,), jnp.float32),
        pltpu.SemaphoreType.DMA,
        pltpu.SemaphoreType.DMA,
    ],
)
def _sc_loss(coord_hbm, span_hbm, lc_hbm, ci_hbm, lcs_hbm, cm_hbm,
             out_hbm, coord_v, span_v, idx_v, gt_v, ci_v, lcs_v, cm_v,
             stage_v, sem, sem2):
    b = lax.axis_index("s")
    c = lax.axis_index("c")
    wid = b * _NC + c
    bt = lax.shift_right_logical(b, 3)
    bs = b & 7

    cp_span = [
        pltpu.async_copy(
            span_hbm.at[pl.ds((b * 2 + c) * _HW, _HW)], span_v, sem2),
        pltpu.async_copy(lcs_hbm.at[pl.ds(b * 2 * _N, 2 * _N)], lcs_v, sem2),
        pltpu.async_copy(ci_hbm.at[bt, :, bs], ci_v, sem2),
    ]
    cp_coord = [
        pltpu.async_copy(
            coord_hbm.at[pl.ds((b * 2 + c) * _HW, _HW)], coord_v, sem),
        pltpu.async_copy(lc_hbm.at[pl.ds(b * 3 * _KC, _KC)], idx_v, sem),
        pltpu.async_copy(
            lc_hbm.at[pl.ds((b * 3 + 1 + c) * _KC, _KC)], gt_v, sem),
        pltpu.async_copy(cm_hbm.at[bt, :, bs], cm_v, sem),
    ]
    for cp in cp_coord:
        cp.wait()

    @plsc.parallel_loop(0, _KC // _L, carry=jnp.zeros((_L,), jnp.float32),
                        unroll=4)
    def acc_c(g, acc):
        p0 = g * _L
        idx = idx_v[pl.ds(p0, _L)]
        gt = gt_v[pl.ds(p0, _L)].astype(jnp.float32)
        pred = plsc.load_gather(coord_v, [idx])
        m = cm_v[lax.shift_right_logical(g, 5), pl.ds((g & 7) * _L, _L)]
        return acc + jnp.abs(pred * m - gt * m)

    msel = jnp.where(c == 0, 1.0, 0.0).astype(jnp.float32)

    for cp in cp_span:
        cp.wait()

    @plsc.parallel_loop(
        0, _N // _L,
        carry=(jnp.zeros((_L,), jnp.float32), jnp.zeros((_L,), jnp.float32)),
        unroll=4)
    def span_accs(g, carry):
        acc_s, acc_m = carry
        nt = lax.shift_right_logical(g, 3)
        nl0 = (g & 7) * _L
        ind = ci_v[nt, pl.ds(nl0, _L)]
        sp = plsc.load_gather(span_v, [ind])
        gt = lcs_v[pl.ds(nt * 256 + c * 128 + nl0, _L)]
        m = cm_v[nt, pl.ds(nl0, _L)]
        return acc_s + jnp.abs(sp * m - gt * m), acc_m + m * msel

    acc_s, acc_m = span_accs

    lane = lax.iota(jnp.int32, _L)
    packed = jnp.where(lane == 0, jnp.sum(acc_c),
                       jnp.where(lane == 1, jnp.sum(acc_s),
                                 jnp.where(lane == 2, jnp.sum(acc_m), 0.0)))
    stage_v[pl.ds(0, _L)] = packed.astype(jnp.float32)
    zeros = jnp.zeros((_L,), jnp.float32)
    for k in range(1, 8):
        stage_v[pl.ds(k * _L, _L)] = zeros
    pltpu.sync_copy(stage_v, out_hbm.at[pl.ds(wid * 128, 128)])


def kernel(coord, span, lc_coords, lc_span, ct_ind, ct_mask):
    lc_lin = (lc_coords.reshape(_B, 8, 128, 4, 3)
              .transpose(0, 4, 1, 3, 2)
              .reshape(_B * 3 * _KC))
    lcs_lin = (lc_span.reshape(_B, 8, 128, 2)
               .transpose(0, 1, 3, 2)
               .reshape(_B * 2 * _N))
    ci_lin = ct_ind.reshape(2, 8, 8, 128).transpose(0, 2, 1, 3)
    cm_lin = ct_mask.reshape(2, 8, 8, 128).transpose(0, 2, 1, 3)
    partials = _sc_loss(
        coord.reshape(_B * 2 * _HW),
        span.reshape(_B * 2 * _HW),
        lc_lin,
        ci_lin,
        lcs_lin,
        cm_lin,
    )
    s = partials.reshape(_NW, 128).sum(axis=0)
    coord_sum = s[0]
    span_sum = s[1]
    mask_sum = s[2]
    coord_loss = coord_sum / (4.0 * mask_sum + _EPS)
    span_loss = span_sum / (2.0 * mask_sum + _EPS)
    return (coord_loss, jnp.array(0.0, dtype=jnp.float32), span_loss)
